# baseline (speedup 1.0000x reference)
import functools

import jax
import jax.numpy as jnp
from jax import lax
from jax.experimental import pallas as pl
from jax.experimental.pallas import tpu as pltpu
from jax.experimental.pallas import tpu_sc as plsc

N = 10000
E = 320000
D = 128

NC = 2
NS = 16
NW = NC * NS

C = 128
CHT = 157
CH0 = 70
CH1 = CHT - CH0
CHM = max(CH0, CH1)
E_PAD = NS * CHT * C
NP = 10240
RPT = NP // NS


def _sc_agg_body(h_hbm, src_hbm, dst_hbm, z2_hbm, z1_hbm,
                 pout_hbm, cout_hbm,
                 src_v, dst_v, rows_v, ones_v, sem, agg_sh, cnt_sh):
    c = lax.axis_index("c")
    s = lax.axis_index("s")
    wid = s * NC + c
    row0 = s * RPT

    pltpu.sync_copy(z2_hbm.at[pl.ds(row0, RPT)], agg_sh.at[pl.ds(row0, RPT)])
    pltpu.sync_copy(z1_hbm.at[pl.ds(row0, RPT)], cnt_sh.at[pl.ds(row0, RPT)])

    pltpu.sync_copy(src_hbm.at[wid], src_v)
    pltpu.sync_copy(dst_hbm.at[wid], dst_v)

    def fill_ones(j, carry):
        ones_v[pl.ds(j * 16, 16)] = jnp.ones((16,), jnp.float32)
        return carry
    lax.fori_loop(0, C // 16, fill_ones, 0)

    plsc.subcore_barrier()

    nch = jnp.where(c == 0, CH0, CH1)

    def edge_chunk(j, carry):
        pltpu.async_copy(h_hbm.at[src_v.at[j]], rows_v, sem).wait()
        pltpu.sync_copy(rows_v, agg_sh.at[dst_v.at[j]], add=True)
        pltpu.sync_copy(ones_v, cnt_sh.at[dst_v.at[j]], add=True)
        return carry
    lax.fori_loop(0, nch, edge_chunk, 0)

    plsc.subcore_barrier()

    pltpu.sync_copy(agg_sh.at[pl.ds(row0, RPT)], pout_hbm.at[c, pl.ds(row0, RPT)])
    pltpu.sync_copy(cnt_sh.at[pl.ds(row0, RPT)], cout_hbm.at[c, pl.ds(row0, RPT)])


@jax.jit
def _sc_agg(h, src3, dst3, z2, z1):
    mesh = plsc.VectorSubcoreMesh(core_axis_name="c", subcore_axis_name="s")
    f = pl.kernel(
        _sc_agg_body,
        out_type=(
            jax.ShapeDtypeStruct((NC, NP, D), jnp.float32),
            jax.ShapeDtypeStruct((NC, NP), jnp.float32),
        ),
        mesh=mesh,
        scratch_types=[
            pltpu.VMEM((CHM, C), jnp.int32),
            pltpu.VMEM((CHM, C), jnp.int32),
            pltpu.VMEM((C, D), jnp.float32),
            pltpu.VMEM((C,), jnp.float32),
            pltpu.SemaphoreType.DMA,
            pltpu.VMEM_SHARED((NP, D), jnp.float32),
            pltpu.VMEM_SHARED((NP,), jnp.float32),
        ],
    )
    return f(h, src3, dst3, z2, z1)


def _tc_dense_body(relu, p_ref, c_ref, h_ref, wl_ref, bl_ref, wr_ref, o_ref):
    agg = p_ref[0] + p_ref[1]
    cnt = c_ref[0] + c_ref[1]
    mean = agg * (1.0 / jnp.maximum(cnt, 1.0))[:, None]
    y = (jnp.dot(mean, wl_ref[...], preferred_element_type=jnp.float32)
         + bl_ref[...]
         + jnp.dot(h_ref[...], wr_ref[...], preferred_element_type=jnp.float32))
    if relu:
        y = jnp.maximum(y, 0.0)
    o_ref[...] = y


@functools.partial(jax.jit, static_argnames=("relu",))
def _tc_dense(p, cnt, h, wl, bl, wr, relu):
    B = 5120
    grid = (NP // B,)
    return pl.pallas_call(
        functools.partial(_tc_dense_body, relu),
        grid=grid,
        in_specs=[
            pl.BlockSpec((NC, B, D), lambda i: (0, i, 0)),
            pl.BlockSpec((NC, B), lambda i: (0, i)),
            pl.BlockSpec((B, D), lambda i: (i, 0)),
            pl.BlockSpec((D, D), lambda i: (0, 0)),
            pl.BlockSpec((1, D), lambda i: (0, 0)),
            pl.BlockSpec((D, D), lambda i: (0, 0)),
        ],
        out_specs=pl.BlockSpec((B, D), lambda i: (i, 0)),
        out_shape=jax.ShapeDtypeStruct((NP, D), jnp.float32),
    )(p, cnt, h, wl, bl, wr)


def kernel(x, edge_index, Wl1, bl1, Wr1, Wl2, bl2, Wr2, Wl3, bl3, Wr3):
    src = edge_index[0]
    dst = edge_index[1]

    def layout(v, fill):
        flat = jnp.concatenate([v, jnp.full((E_PAD - E,), fill, jnp.int32)])
        ch = flat.reshape(NS * CHT, C)
        a = jnp.pad(ch[:NS * CH0].reshape(NS, CH0, C),
                    ((0, 0), (0, CHM - CH0), (0, 0)), constant_values=fill)
        b = ch[NS * CH0:].reshape(NS, CH1, C)
        return jnp.stack([a, b], axis=1).reshape(NW, CHM, C)

    src3 = layout(src, 0)
    dst3 = layout(dst, N)
    z2 = jnp.zeros((NP, D), jnp.float32)
    z1 = jnp.zeros((NP,), jnp.float32)

    h = jnp.pad(x, ((0, NP - N), (0, 0)))
    for Wl, bl, Wr, relu in (
        (Wl1, bl1, Wr1, True),
        (Wl2, bl2, Wr2, True),
        (Wl3, bl3, Wr3, False),
    ):
        p, cnt = _sc_agg(h, src3, dst3, z2, z1)
        h = _tc_dense(p, cnt, h, Wl, bl.reshape(1, D), Wr, relu)
    return h[:N]


# speedup vs baseline: 1.0479x; 1.0479x over previous
import functools

import jax
import jax.numpy as jnp
from jax import lax
from jax.experimental import pallas as pl
from jax.experimental.pallas import tpu as pltpu
from jax.experimental.pallas import tpu_sc as plsc

N = 10000
E = 320000
D = 128

NC = 2
NS = 16
NW = NC * NS

C = 128
CHT = 157
CH0 = 68
CH1 = CHT - CH0
CHM = max(CH0, CH1)
E_PAD = NS * CHT * C
NP = 10240
RPT = NP // NS


def _sc_agg_body(h_hbm, src_hbm, dst_hbm, z2_hbm, z1_hbm,
                 pout_hbm, cout_hbm,
                 src_v, dst_v, rows_v, ones_v, sem, agg_sh, cnt_sh):
    c = lax.axis_index("c")
    s = lax.axis_index("s")
    wid = s * NC + c
    row0 = s * RPT

    pltpu.sync_copy(z2_hbm.at[pl.ds(row0, RPT)], agg_sh.at[pl.ds(row0, RPT)])
    pltpu.sync_copy(z1_hbm.at[pl.ds(row0, RPT)], cnt_sh.at[pl.ds(row0, RPT)])

    pltpu.sync_copy(src_hbm.at[wid], src_v)
    pltpu.sync_copy(dst_hbm.at[wid], dst_v)

    def fill_ones(j, carry):
        ones_v[pl.ds(j * 16, 16)] = jnp.ones((16,), jnp.float32)
        return carry
    lax.fori_loop(0, C // 16, fill_ones, 0)

    plsc.subcore_barrier()

    nch = jnp.where(c == 0, CH0, CH1)

    def edge_chunk(j, carry):
        pltpu.async_copy(h_hbm.at[src_v.at[j]], rows_v, sem).wait()
        pltpu.sync_copy(rows_v, agg_sh.at[dst_v.at[j]], add=True)
        pltpu.sync_copy(ones_v, cnt_sh.at[dst_v.at[j]], add=True)
        return carry
    lax.fori_loop(0, nch, edge_chunk, 0)

    plsc.subcore_barrier()

    pltpu.sync_copy(agg_sh.at[pl.ds(row0, RPT)], pout_hbm.at[c, pl.ds(row0, RPT)])
    pltpu.sync_copy(cnt_sh.at[pl.ds(row0, RPT)], cout_hbm.at[c, pl.ds(row0, RPT)])


@jax.jit
def _sc_agg(h, src3, dst3, z2, z1):
    mesh = plsc.VectorSubcoreMesh(core_axis_name="c", subcore_axis_name="s")
    f = pl.kernel(
        _sc_agg_body,
        out_type=(
            jax.ShapeDtypeStruct((NC, NP, D), jnp.float32),
            jax.ShapeDtypeStruct((NC, NP), jnp.float32),
        ),
        mesh=mesh,
        scratch_types=[
            pltpu.VMEM((CHM, C), jnp.int32),
            pltpu.VMEM((CHM, C), jnp.int32),
            pltpu.VMEM((C, D), jnp.float32),
            pltpu.VMEM((C,), jnp.float32),
            pltpu.SemaphoreType.DMA,
            pltpu.VMEM_SHARED((NP, D), jnp.float32),
            pltpu.VMEM_SHARED((NP,), jnp.float32),
        ],
    )
    return f(h, src3, dst3, z2, z1)


def _tc_dense_body(relu, p_ref, c_ref, h_ref, wl_ref, bl_ref, wr_ref, o_ref):
    agg = p_ref[0] + p_ref[1]
    cnt = c_ref[0] + c_ref[1]
    mean = agg * (1.0 / jnp.maximum(cnt, 1.0))[:, None]
    y = (jnp.dot(mean, wl_ref[...], preferred_element_type=jnp.float32)
         + bl_ref[...]
         + jnp.dot(h_ref[...], wr_ref[...], preferred_element_type=jnp.float32))
    if relu:
        y = jnp.maximum(y, 0.0)
    o_ref[...] = y


@functools.partial(jax.jit, static_argnames=("relu",))
def _tc_dense(p, cnt, h, wl, bl, wr, relu):
    B = 5120
    grid = (NP // B,)
    return pl.pallas_call(
        functools.partial(_tc_dense_body, relu),
        grid=grid,
        in_specs=[
            pl.BlockSpec((NC, B, D), lambda i: (0, i, 0)),
            pl.BlockSpec((NC, B), lambda i: (0, i)),
            pl.BlockSpec((B, D), lambda i: (i, 0)),
            pl.BlockSpec((D, D), lambda i: (0, 0)),
            pl.BlockSpec((1, D), lambda i: (0, 0)),
            pl.BlockSpec((D, D), lambda i: (0, 0)),
        ],
        out_specs=pl.BlockSpec((B, D), lambda i: (i, 0)),
        out_shape=jax.ShapeDtypeStruct((NP, D), jnp.float32),
    )(p, cnt, h, wl, bl, wr)


def kernel(x, edge_index, Wl1, bl1, Wr1, Wl2, bl2, Wr2, Wl3, bl3, Wr3):
    src = edge_index[0]
    dst = edge_index[1]

    def layout(v, fill):
        flat = jnp.concatenate([v, jnp.full((E_PAD - E,), fill, jnp.int32)])
        ch = flat.reshape(NS * CHT, C)
        a = jnp.pad(ch[:NS * CH0].reshape(NS, CH0, C),
                    ((0, 0), (0, CHM - CH0), (0, 0)), constant_values=fill)
        b = ch[NS * CH0:].reshape(NS, CH1, C)
        return jnp.stack([a, b], axis=1).reshape(NW, CHM, C)

    src3 = layout(src, 0)
    dst3 = layout(dst, N)
    z2 = jnp.zeros((NP, D), jnp.float32)
    z1 = jnp.zeros((NP,), jnp.float32)

    h = jnp.pad(x, ((0, NP - N), (0, 0)))
    for Wl, bl, Wr, relu in (
        (Wl1, bl1, Wr1, True),
        (Wl2, bl2, Wr2, True),
        (Wl3, bl3, Wr3, False),
    ):
        p, cnt = _sc_agg(h, src3, dst3, z2, z1)
        h = _tc_dense(p, cnt, h, Wl, bl.reshape(1, D), Wr, relu)
    return h[:N]
